# Initial kernel scaffold; baseline (speedup 1.0000x reference)
import jax
import jax.numpy as jnp
from jax.experimental import pallas as pl


def kernel(x, table):
    raise NotImplementedError("write your pallas kernel here")


# speedup vs baseline: 3.1668x; 3.1668x over previous
import math

import jax
import jax.numpy as jnp
from jax import lax
from jax.experimental import pallas as pl
from jax.experimental.pallas import tpu as pltpu
from jax.experimental.pallas import tpu_sc as plsc

D_MODEL = 1024
SEQ_LEN = 4096
BATCH = 4

NC = 2
NS = 16
NW = NC * NS
POS_PER_W = SEQ_LEN // NW
CHUNK = 32
NCHUNK = POS_PER_W // CHUNK
LANES = 16
VECS_PER_ROW = D_MODEL // LANES


def _pe_table(max_len, d_model):
    pos = jnp.arange(max_len, dtype=jnp.float32)[:, None]
    div = jnp.exp(
        jnp.arange(0, d_model, 2, dtype=jnp.float32)
        * -(math.log(10000.0) / d_model)
    )
    pe = jnp.zeros((max_len, d_model), dtype=jnp.float32)
    pe = pe.at[:, 0::2].set(jnp.sin(pos * div))
    pe = pe.at[:, 1::2].set(jnp.cos(pos * div))
    return pe


def _body(idx_hbm, table_hbm, pe_hbm, out_hbm, idx_v, pe_v, rows_v, sem):
    wid = lax.axis_index("s") * NC + lax.axis_index("c")
    l0 = wid * POS_PER_W

    for b in range(BATCH):
        pltpu.sync_copy(
            idx_hbm.at[pl.ds(b * SEQ_LEN + l0, POS_PER_W)],
            idx_v.at[pl.ds(b * POS_PER_W, POS_PER_W)],
        )

    for c in range(NCHUNK):
        pltpu.sync_copy(pe_hbm.at[pl.ds(l0 + c * CHUNK, CHUNK)], pe_v)
        for b in range(BATCH):
            idx_slice = idx_v.at[pl.ds(b * POS_PER_W + c * CHUNK, CHUNK)]
            pltpu.async_copy(table_hbm.at[idx_slice], rows_v, sem).wait()

            def add_pe(i):
                r = i // VECS_PER_ROW
                col = (i % VECS_PER_ROW) * LANES
                v = pe_v[r, pl.ds(col, LANES)]
                plsc.addupdate(rows_v.at[r, pl.ds(col, LANES)], v)

            plsc.parallel_loop(0, CHUNK * VECS_PER_ROW, unroll=8)(add_pe)

            pltpu.sync_copy(
                rows_v,
                out_hbm.at[pl.ds(b * SEQ_LEN + l0 + c * CHUNK, CHUNK)],
            )


def kernel(x, table):
    idx = x.reshape(BATCH * SEQ_LEN).astype(jnp.int32)
    pe = _pe_table(SEQ_LEN, D_MODEL)

    run = pl.kernel(
        _body,
        out_type=jax.ShapeDtypeStruct((BATCH * SEQ_LEN, D_MODEL), jnp.float32),
        mesh=plsc.VectorSubcoreMesh(core_axis_name="c", subcore_axis_name="s"),
        scratch_types=[
            pltpu.VMEM((BATCH * POS_PER_W,), jnp.int32),
            pltpu.VMEM((CHUNK, D_MODEL), jnp.float32),
            pltpu.VMEM((CHUNK, D_MODEL), jnp.float32),
            pltpu.SemaphoreType.DMA,
        ],
    )
    out = run(idx, table, pe)
    return out.reshape(BATCH, SEQ_LEN, D_MODEL)


# speedup vs baseline: 3.7756x; 1.1922x over previous
import math

import jax
import jax.numpy as jnp
from jax import lax
from jax.experimental import pallas as pl
from jax.experimental.pallas import tpu as pltpu
from jax.experimental.pallas import tpu_sc as plsc

D_MODEL = 1024
SEQ_LEN = 4096
BATCH = 4

NC = 2
NS = 16
NW = NC * NS
POS_PER_W = SEQ_LEN // NW
CHUNK = 32
NCHUNK = POS_PER_W // CHUNK
NSTEP = NCHUNK * BATCH
LANES = 16
VECS_PER_ROW = D_MODEL // LANES


def _pe_table(max_len, d_model):
    pos = jnp.arange(max_len, dtype=jnp.float32)[:, None]
    div = jnp.exp(
        jnp.arange(0, d_model, 2, dtype=jnp.float32)
        * -(math.log(10000.0) / d_model)
    )
    pe = jnp.zeros((max_len, d_model), dtype=jnp.float32)
    pe = pe.at[:, 0::2].set(jnp.sin(pos * div))
    pe = pe.at[:, 1::2].set(jnp.cos(pos * div))
    return pe


def _body(idx_hbm, table_hbm, pe_hbm, out_hbm,
          idx_v, pe_v, rows_v, sem_g0, sem_g1, sem_o0, sem_o1):
    wid = lax.axis_index("s") * NC + lax.axis_index("c")
    l0 = wid * POS_PER_W
    sem_g = (sem_g0, sem_g1)
    sem_o = (sem_o0, sem_o1)

    for b in range(BATCH):
        pltpu.sync_copy(
            idx_hbm.at[pl.ds(b * SEQ_LEN + l0, POS_PER_W)],
            idx_v.at[pl.ds(b * POS_PER_W, POS_PER_W)],
        )

    def gather(k, buf):
        c, b = k // BATCH, k % BATCH
        idx_slice = idx_v.at[pl.ds(b * POS_PER_W + c * CHUNK, CHUNK)]
        return pltpu.async_copy(table_hbm.at[idx_slice], rows_v.at[buf],
                                sem_g[buf])

    def scatter(k, buf):
        c, b = k // BATCH, k % BATCH
        return pltpu.async_copy(
            rows_v.at[buf],
            out_hbm.at[pl.ds(b * SEQ_LEN + l0 + c * CHUNK, CHUNK)],
            sem_o[buf])

    def add_pe(buf):
        def body(i):
            r = i // VECS_PER_ROW
            col = (i % VECS_PER_ROW) * LANES
            v = pe_v[r, pl.ds(col, LANES)]
            plsc.addupdate(rows_v.at[buf, r, pl.ds(col, LANES)], v)
        plsc.parallel_loop(0, CHUNK * VECS_PER_ROW, unroll=8)(body)

    g = [None, None]
    o = [None, None]
    g[0] = gather(0, 0)
    for k in range(NSTEP):
        buf = k % 2
        if k % BATCH == 0:
            pltpu.sync_copy(
                pe_hbm.at[pl.ds(l0 + (k // BATCH) * CHUNK, CHUNK)], pe_v)
        g[buf].wait()
        if k + 1 < NSTEP:
            if o[1 - buf] is not None:
                o[1 - buf].wait()
            g[1 - buf] = gather(k + 1, 1 - buf)
        add_pe(buf)
        o[buf] = scatter(k, buf)
    o[0].wait()
    o[1].wait()


def kernel(x, table):
    idx = x.reshape(BATCH * SEQ_LEN).astype(jnp.int32)
    pe = _pe_table(SEQ_LEN, D_MODEL)

    run = pl.kernel(
        _body,
        out_type=jax.ShapeDtypeStruct((BATCH * SEQ_LEN, D_MODEL), jnp.float32),
        mesh=plsc.VectorSubcoreMesh(core_axis_name="c", subcore_axis_name="s"),
        scratch_types=[
            pltpu.VMEM((BATCH * POS_PER_W,), jnp.int32),
            pltpu.VMEM((CHUNK, D_MODEL), jnp.float32),
            pltpu.VMEM((2, CHUNK, D_MODEL), jnp.float32),
            pltpu.SemaphoreType.DMA,
            pltpu.SemaphoreType.DMA,
            pltpu.SemaphoreType.DMA,
            pltpu.SemaphoreType.DMA,
        ],
    )
    out = run(idx, table, pe)
    return out.reshape(BATCH, SEQ_LEN, D_MODEL)


# speedup vs baseline: 7.0657x; 1.8714x over previous
import functools
import math

import jax
import jax.numpy as jnp
import numpy as np
from jax import lax
from jax.experimental import pallas as pl
from jax.experimental.pallas import tpu as pltpu
from jax.experimental.pallas import tpu_sc as plsc

D_MODEL = 1024
SEQ_LEN = 4096
BATCH = 4

NC = 2
NS = 16
NW = NC * NS
POS_PER_W = SEQ_LEN // NW
CHUNK = 32
NCHUNK = POS_PER_W // CHUNK
NSTEP = NCHUNK * BATCH
LANES = 16
VECS_PER_ROW = D_MODEL // LANES


@functools.lru_cache(maxsize=None)
def _pe_table(max_len, d_model):
    pos = np.arange(max_len, dtype=np.float32)[:, None]
    div = np.exp(
        np.arange(0, d_model, 2, dtype=np.float32)
        * -(math.log(10000.0) / d_model)
    )
    pe = np.zeros((max_len, d_model), dtype=np.float32)
    pe[:, 0::2] = np.sin(pos * div)
    pe[:, 1::2] = np.cos(pos * div)
    return jnp.asarray(pe)


def _body(idx_hbm, table_hbm, pe_hbm, out_hbm,
          idx_v, pe_v, rows_v, sem_g0, sem_g1, sem_o0, sem_o1):
    wid = lax.axis_index("s") * NC + lax.axis_index("c")
    l0 = wid * POS_PER_W
    sem_g = (sem_g0, sem_g1)
    sem_o = (sem_o0, sem_o1)

    for b in range(BATCH):
        pltpu.sync_copy(
            idx_hbm.at[pl.ds(b * SEQ_LEN + l0, POS_PER_W)],
            idx_v.at[pl.ds(b * POS_PER_W, POS_PER_W)],
        )

    def gather(k, buf):
        c, b = k // BATCH, k % BATCH
        idx_slice = idx_v.at[pl.ds(b * POS_PER_W + c * CHUNK, CHUNK)]
        return pltpu.async_copy(table_hbm.at[idx_slice], rows_v.at[buf],
                                sem_g[buf])

    def scatter(k, buf):
        c, b = k // BATCH, k % BATCH
        return pltpu.async_copy(
            rows_v.at[buf],
            out_hbm.at[pl.ds(b * SEQ_LEN + l0 + c * CHUNK, CHUNK)],
            sem_o[buf])

    def add_pe(buf):
        def body(i):
            r = i // VECS_PER_ROW
            col = (i % VECS_PER_ROW) * LANES
            v = pe_v[r, pl.ds(col, LANES)]
            plsc.addupdate(rows_v.at[buf, r, pl.ds(col, LANES)], v)
        plsc.parallel_loop(0, CHUNK * VECS_PER_ROW, unroll=8)(body)

    g = [None, None]
    o = [None, None]
    g[0] = gather(0, 0)
    for k in range(NSTEP):
        buf = k % 2
        if k % BATCH == 0:
            pltpu.sync_copy(
                pe_hbm.at[pl.ds(l0 + (k // BATCH) * CHUNK, CHUNK)], pe_v)
        g[buf].wait()
        if k + 1 < NSTEP:
            if o[1 - buf] is not None:
                o[1 - buf].wait()
            g[1 - buf] = gather(k + 1, 1 - buf)
        add_pe(buf)
        o[buf] = scatter(k, buf)
    o[0].wait()
    o[1].wait()


def kernel(x, table):
    idx = x.reshape(BATCH * SEQ_LEN).astype(jnp.int32)
    pe = _pe_table(SEQ_LEN, D_MODEL)

    run = pl.kernel(
        _body,
        out_type=jax.ShapeDtypeStruct((BATCH * SEQ_LEN, D_MODEL), jnp.float32),
        mesh=plsc.VectorSubcoreMesh(core_axis_name="c", subcore_axis_name="s"),
        scratch_types=[
            pltpu.VMEM((BATCH * POS_PER_W,), jnp.int32),
            pltpu.VMEM((CHUNK, D_MODEL), jnp.float32),
            pltpu.VMEM((2, CHUNK, D_MODEL), jnp.float32),
            pltpu.SemaphoreType.DMA,
            pltpu.SemaphoreType.DMA,
            pltpu.SemaphoreType.DMA,
            pltpu.SemaphoreType.DMA,
        ],
    )
    out = run(idx, table, pe)
    return out.reshape(BATCH, SEQ_LEN, D_MODEL)


# speedup vs baseline: 7.1288x; 1.0089x over previous
import functools
import math

import jax
import jax.numpy as jnp
import numpy as np
from jax import lax
from jax.experimental import pallas as pl
from jax.experimental.pallas import tpu as pltpu
from jax.experimental.pallas import tpu_sc as plsc

D_MODEL = 1024
SEQ_LEN = 4096
BATCH = 4

NC = 2
NS = 16
NW = NC * NS
POS_PER_W = SEQ_LEN // NW
CHUNK = 32
NCHUNK = POS_PER_W // CHUNK
NSTEP = NCHUNK * BATCH
LANES = 16
VECS_PER_ROW = D_MODEL // LANES


@functools.lru_cache(maxsize=None)
def _pe_table(max_len, d_model):
    pos = np.arange(max_len, dtype=np.float32)[:, None]
    div = np.exp(
        np.arange(0, d_model, 2, dtype=np.float32)
        * -(math.log(10000.0) / d_model)
    )
    pe = np.zeros((max_len, d_model), dtype=np.float32)
    pe[:, 0::2] = np.sin(pos * div)
    pe[:, 1::2] = np.cos(pos * div)
    return jnp.asarray(pe.reshape(-1))


def _body(idx_hbm, table_hbm, pe_hbm, out_hbm,
          idx_v, pe_v, rows_v, sem_g0, sem_g1, sem_o0, sem_o1):
    wid = lax.axis_index("s") * NC + lax.axis_index("c")
    l0 = wid * POS_PER_W
    sem_g = (sem_g0, sem_g1)
    sem_o = (sem_o0, sem_o1)

    for b in range(BATCH):
        pltpu.sync_copy(
            idx_hbm.at[pl.ds(b * SEQ_LEN + l0, POS_PER_W)],
            idx_v.at[pl.ds(b * POS_PER_W, POS_PER_W)],
        )

    def gather(k, buf):
        c, b = k // BATCH, k % BATCH
        idx_slice = idx_v.at[pl.ds(b * POS_PER_W + c * CHUNK, CHUNK)]
        return pltpu.async_copy(table_hbm.at[idx_slice], rows_v.at[buf],
                                sem_g[buf])

    def scatter(k, buf):
        c, b = k // BATCH, k % BATCH
        return pltpu.async_copy(
            rows_v.at[buf],
            out_hbm.at[pl.ds(b * SEQ_LEN + l0 + c * CHUNK, CHUNK)],
            sem_o[buf])

    def add_pe(buf):
        def body(i):
            r = i // VECS_PER_ROW
            col = (i % VECS_PER_ROW) * LANES
            v = pe_v[pl.ds(i * LANES, LANES)]
            plsc.addupdate(rows_v.at[buf, r, pl.ds(col, LANES)], v)
        plsc.parallel_loop(0, CHUNK * VECS_PER_ROW, unroll=8)(body)

    g = [None, None]
    o = [None, None]
    g[0] = gather(0, 0)
    for k in range(NSTEP):
        buf = k % 2
        if k % BATCH == 0:
            pltpu.sync_copy(
                pe_hbm.at[pl.ds((l0 + (k // BATCH) * CHUNK) * D_MODEL,
                                CHUNK * D_MODEL)],
                pe_v)
        g[buf].wait()
        if k + 1 < NSTEP:
            if o[1 - buf] is not None:
                o[1 - buf].wait()
            g[1 - buf] = gather(k + 1, 1 - buf)
        add_pe(buf)
        o[buf] = scatter(k, buf)
    o[0].wait()
    o[1].wait()


def kernel(x, table):
    idx = x.reshape(BATCH * SEQ_LEN).astype(jnp.int32)
    pe = _pe_table(SEQ_LEN, D_MODEL)

    run = pl.kernel(
        _body,
        out_type=jax.ShapeDtypeStruct((BATCH * SEQ_LEN, D_MODEL), jnp.float32),
        mesh=plsc.VectorSubcoreMesh(core_axis_name="c", subcore_axis_name="s"),
        scratch_types=[
            pltpu.VMEM((BATCH * POS_PER_W,), jnp.int32),
            pltpu.VMEM((CHUNK * D_MODEL,), jnp.float32),
            pltpu.VMEM((2, CHUNK, D_MODEL), jnp.float32),
            pltpu.SemaphoreType.DMA,
            pltpu.SemaphoreType.DMA,
            pltpu.SemaphoreType.DMA,
            pltpu.SemaphoreType.DMA,
        ],
    )
    out = run(idx, table, pe)
    return out.reshape(BATCH, SEQ_LEN, D_MODEL)
